# Initial kernel scaffold; baseline (speedup 1.0000x reference)
import functools

import jax
import jax.numpy as jnp
from jax import lax
from jax.experimental import pallas as pl
from jax.experimental.pallas import tpu as pltpu
from jax.experimental.pallas import tpu_sc as plsc

_VOCAB = 1000000
_DIM = 32
_B = 16384
_L = 50
_N = _B * _L

_NC = 2
_NS = 16
_NW = _NC * _NS
_PER_W = _N // _NW
_CH = 1024
_NCH = _PER_W // _CH


def _make_kernel():
    mesh = plsc.VectorSubcoreMesh(core_axis_name="c", subcore_axis_name="s")

    @functools.partial(
        pl.kernel,
        mesh=mesh,
        out_type=jax.ShapeDtypeStruct((_N, 2, _DIM), jnp.float32),
        scratch_types=[
            pltpu.VMEM((_CH,), jnp.int32),
            pltpu.VMEM((_CH,), jnp.int32),
            pltpu.VMEM((_CH, _DIM), jnp.float32),
            pltpu.VMEM((_CH, _DIM), jnp.float32),
            pltpu.SemaphoreType.DMA,
            pltpu.SemaphoreType.DMA,
        ],
    )
    def emb_kernel(ids_t_hbm, ids_p_hbm, wt_hbm, wp_hbm, out_hbm,
                   idx_t, idx_p, rows_t, rows_p, sem_t, sem_p):
        wid = lax.axis_index("s") * _NC + lax.axis_index("c")
        wbase = wid * _PER_W

        def body(i, carry):
            base = wbase + i * _CH
            pltpu.sync_copy(ids_t_hbm.at[pl.ds(base, _CH)], idx_t)
            pltpu.sync_copy(ids_p_hbm.at[pl.ds(base, _CH)], idx_p)
            ct = pltpu.async_copy(wt_hbm.at[idx_t], rows_t, sem_t)
            cp = pltpu.async_copy(wp_hbm.at[idx_p], rows_p, sem_p)
            ct.wait()
            cp.wait()
            pltpu.sync_copy(rows_t, out_hbm.at[pl.ds(base, _CH), 0])
            pltpu.sync_copy(rows_p, out_hbm.at[pl.ds(base, _CH), 1])
            return carry

        lax.fori_loop(0, _NCH, body, 0)

    return emb_kernel


_EMB_KERNEL = _make_kernel()


def kernel(input_ids, tokens_pretrain, W_trainable, W_pretrained):
    ids_t = input_ids.reshape(_N).astype(jnp.int32)
    ids_p = tokens_pretrain.reshape(_N).astype(jnp.int32)
    out = _EMB_KERNEL(ids_t, ids_p, W_trainable, W_pretrained)
    return out.reshape(_B, _L, 2 * _DIM)


# speedup vs baseline: 2.0605x; 2.0605x over previous
import functools

import jax
import jax.numpy as jnp
from jax import lax
from jax.experimental import pallas as pl
from jax.experimental.pallas import tpu as pltpu
from jax.experimental.pallas import tpu_sc as plsc

_VOCAB = 1000000
_DIM = 32
_B = 16384
_L = 50
_N = _B * _L

_NC = 2
_NS = 16
_NW = _NC * _NS
_PER_W = _N // _NW
_CH = 1024
_NCH = _PER_W // _CH


@functools.cache
def _make_kernel():
    mesh = plsc.VectorSubcoreMesh(core_axis_name="c", subcore_axis_name="s")

    @functools.partial(
        pl.kernel,
        mesh=mesh,
        compiler_params=pltpu.CompilerParams(use_tc_tiling_on_sc=False),
        out_type=jax.ShapeDtypeStruct((_N, 2, _DIM), jnp.float32),
        scratch_types=[
            pltpu.VMEM((_CH,), jnp.int32),
            pltpu.VMEM((_CH,), jnp.int32),
            pltpu.VMEM((_CH, _DIM), jnp.float32),
            pltpu.VMEM((_CH, _DIM), jnp.float32),
            pltpu.SemaphoreType.DMA,
            pltpu.SemaphoreType.DMA,
        ],
    )
    def emb_kernel(ids_t_hbm, ids_p_hbm, wt_hbm, wp_hbm, out_hbm,
                   idx_t, idx_p, rows_t, rows_p, sem_t, sem_p):
        wid = lax.axis_index("s") * _NC + lax.axis_index("c")
        wbase = wid * _PER_W

        def body(i, carry):
            base = wbase + i * _CH
            pltpu.sync_copy(ids_t_hbm.at[pl.ds(base, _CH)], idx_t)
            pltpu.sync_copy(ids_p_hbm.at[pl.ds(base, _CH)], idx_p)
            ct = pltpu.async_copy(wt_hbm.at[idx_t], rows_t, sem_t)
            cp = pltpu.async_copy(wp_hbm.at[idx_p], rows_p, sem_p)
            ct.wait()
            cp.wait()
            pltpu.sync_copy(rows_t, out_hbm.at[pl.ds(base, _CH), 0])
            pltpu.sync_copy(rows_p, out_hbm.at[pl.ds(base, _CH), 1])
            return carry

        lax.fori_loop(0, _NCH, body, 0)

    return emb_kernel


def kernel(input_ids, tokens_pretrain, W_trainable, W_pretrained):
    ids_t = input_ids.reshape(_N).astype(jnp.int32)
    ids_p = tokens_pretrain.reshape(_N).astype(jnp.int32)
    out = _make_kernel()(ids_t, ids_p, W_trainable, W_pretrained)
    return out.reshape(_B, _L, 2 * _DIM)


# speedup vs baseline: 2.1223x; 1.0300x over previous
import functools

import jax
import jax.numpy as jnp
from jax import lax
from jax.experimental import pallas as pl
from jax.experimental.pallas import tpu as pltpu
from jax.experimental.pallas import tpu_sc as plsc

_VOCAB = 1000000
_DIM = 32
_B = 16384
_L = 50
_N = _B * _L

_NC = 2
_NS = 16
_NW = _NC * _NS
_PER_W = _N // _NW
_CH = 512
_NCH = _PER_W // _CH


@functools.cache
def _make_kernel():
    mesh = plsc.VectorSubcoreMesh(core_axis_name="c", subcore_axis_name="s")

    @functools.partial(
        pl.kernel,
        mesh=mesh,
        compiler_params=pltpu.CompilerParams(use_tc_tiling_on_sc=False),
        out_type=jax.ShapeDtypeStruct((_N, 2, _DIM), jnp.float32),
        scratch_types=[
            pltpu.VMEM((_PER_W,), jnp.int32),
            pltpu.VMEM((_PER_W,), jnp.int32),
            pltpu.VMEM((2, _CH, _DIM), jnp.float32),
            pltpu.VMEM((2, _CH, _DIM), jnp.float32),
            pltpu.SemaphoreType.DMA,
            pltpu.SemaphoreType.DMA,
            pltpu.SemaphoreType.DMA,
            pltpu.SemaphoreType.DMA,
        ],
    )
    def emb_kernel(ids_t_hbm, ids_p_hbm, wt_hbm, wp_hbm, out_hbm,
                   idx_t, idx_p, rows_t, rows_p,
                   semg0, semg1, semw0, semw1):
        wid = lax.axis_index("s") * _NC + lax.axis_index("c")
        wbase = wid * _PER_W
        semg = (semg0, semg1)
        semw = (semw0, semw1)

        def fire_g(i, s):
            base = i * _CH
            pltpu.async_copy(wt_hbm.at[idx_t.at[pl.ds(base, _CH)]],
                             rows_t.at[s], semg[s])
            pltpu.async_copy(wp_hbm.at[idx_p.at[pl.ds(base, _CH)]],
                             rows_p.at[s], semg[s])

        def wait_g(s):
            pltpu.make_async_copy(wt_hbm.at[idx_t.at[pl.ds(0, _CH)]],
                                  rows_t.at[s], semg[s]).wait()
            pltpu.make_async_copy(wp_hbm.at[idx_p.at[pl.ds(0, _CH)]],
                                  rows_p.at[s], semg[s]).wait()

        def fire_w(i, s):
            gbase = wbase + i * _CH
            pltpu.async_copy(rows_t.at[s], out_hbm.at[pl.ds(gbase, _CH), 0],
                             semw[s])
            pltpu.async_copy(rows_p.at[s], out_hbm.at[pl.ds(gbase, _CH), 1],
                             semw[s])

        def wait_w(s):
            pltpu.make_async_copy(rows_t.at[s], out_hbm.at[pl.ds(0, _CH), 0],
                                  semw[s]).wait()
            pltpu.make_async_copy(rows_p.at[s], out_hbm.at[pl.ds(0, _CH), 1],
                                  semw[s]).wait()

        pltpu.sync_copy(ids_t_hbm.at[pl.ds(wbase, _PER_W)], idx_t)
        pltpu.sync_copy(ids_p_hbm.at[pl.ds(wbase, _PER_W)], idx_p)

        fire_g(0, 0)
        fire_g(1, 1)
        wait_g(0)
        fire_w(0, 0)

        def body(g, carry):
            i0 = 2 * g
            wait_w(0)
            fire_g(i0, 0)
            wait_g(1)
            fire_w(i0 - 1, 1)
            wait_w(1)
            fire_g(i0 + 1, 1)
            wait_g(0)
            fire_w(i0, 0)
            return carry

        lax.fori_loop(1, _NCH // 2, body, 0)

        wait_g(1)
        fire_w(_NCH - 1, 1)
        wait_w(0)
        wait_w(1)

    return emb_kernel


def kernel(input_ids, tokens_pretrain, W_trainable, W_pretrained):
    ids_t = input_ids.reshape(_N).astype(jnp.int32)
    ids_p = tokens_pretrain.reshape(_N).astype(jnp.int32)
    out = _make_kernel()(ids_t, ids_p, W_trainable, W_pretrained)
    return out.reshape(_B, _L, 2 * _DIM)
